# Initial kernel scaffold; baseline (speedup 1.0000x reference)
import jax
import jax.numpy as jnp
from jax.experimental import pallas as pl


def kernel(node_features, src_idx, dst_idx, params):
    raise NotImplementedError("write your pallas kernel here")


# speedup vs baseline: 2.3898x; 2.3898x over previous
import functools

import jax
import jax.numpy as jnp
from jax import lax
from jax.experimental import pallas as pl
from jax.experimental.pallas import tpu as pltpu
from jax.experimental.pallas import tpu_sc as plsc

NN = 10000
NH = 2048
NE = 320000
D = 128
FF = 4 * D
EPS = 1e-5

NW = 32
K = 128
EP = 10240
CH = EP // K
NE_PAD = NW * EP
ACC_N = 10240
ACC_H = 2560

_MESH = plsc.VectorSubcoreMesh(core_axis_name="c", subcore_axis_name="s",
                               num_cores=2, num_subcores=16)


def _zero_rows(zeros_hbm, acc, base, nrows, width):
    off = 0
    while off < nrows:
        step = min(K, nrows - off)
        pltpu.sync_copy(zeros_hbm.at[pl.ds(0, step)],
                        acc.at[pl.ds(base + off, step)])
        off += step


def _make_deg(n_out, acc_rows):
    @functools.partial(
        pl.kernel,
        mesh=_MESH,
        out_type=jax.ShapeDtypeStruct((2, n_out, D), jnp.float32),
        scratch_types=[
            pltpu.VMEM((CH, K), jnp.int32),
            pltpu.VMEM((K, D), jnp.float32),
            pltpu.VMEM_SHARED((acc_rows, D), jnp.float32),
        ],
    )
    def deg(sidx_hbm, zeros_hbm, ones_hbm, out_hbm, sidx_v, ones_v, acc):
        c = lax.axis_index("c")
        s = lax.axis_index("s")
        wid = s * 2 + c
        _zero_rows(zeros_hbm, acc, s * (acc_rows // 16), acc_rows // 16, D)
        pltpu.sync_copy(ones_hbm, ones_v)
        pltpu.sync_copy(sidx_hbm.at[wid], sidx_v)
        plsc.subcore_barrier()

        def body(ci, carry):
            pltpu.sync_copy(ones_v, acc.at[sidx_v.at[ci]], add=True)
            return carry

        lax.fori_loop(0, CH, body, 0)
        plsc.subcore_barrier()
        rw = n_out // 16
        pltpu.sync_copy(acc.at[pl.ds(s * rw, rw)],
                        out_hbm.at[c, pl.ds(s * rw, rw)])

    return deg


_deg_n = _make_deg(ACC_N, ACC_N)
_deg_h = _make_deg(NH, ACC_H)


def _make_conv(n_out, acc_rows):
    @functools.partial(
        pl.kernel,
        mesh=_MESH,
        out_type=jax.ShapeDtypeStruct((2, n_out, D), jnp.float32),
        scratch_types=[
            pltpu.VMEM((CH, K), jnp.int32),
            pltpu.VMEM((CH, K), jnp.int32),
            pltpu.VMEM((K, D), jnp.float32),
            pltpu.VMEM_SHARED((acc_rows, D), jnp.float32),
            pltpu.SemaphoreType.DMA,
        ],
    )
    def conv(x_hbm, gidx_hbm, sidx_hbm, zeros_hbm,
             out_hbm, gidx_v, sidx_v, rows_v, acc, sem):
        c = lax.axis_index("c")
        s = lax.axis_index("s")
        wid = s * 2 + c
        _zero_rows(zeros_hbm, acc, s * (acc_rows // 16), acc_rows // 16, D)
        pltpu.sync_copy(gidx_hbm.at[wid], gidx_v)
        pltpu.sync_copy(sidx_hbm.at[wid], sidx_v)
        plsc.subcore_barrier()

        def body(ci, carry):
            pltpu.async_copy(x_hbm.at[gidx_v.at[ci]], rows_v, sem).wait()
            pltpu.sync_copy(rows_v, acc.at[sidx_v.at[ci]], add=True)
            return carry

        lax.fori_loop(0, CH, body, 0)
        plsc.subcore_barrier()
        rw = n_out // 16
        pltpu.sync_copy(acc.at[pl.ds(s * rw, rw)],
                        out_hbm.at[c, pl.ds(s * rw, rw)])

    return conv


_conv_to_hyper = _make_conv(NH, ACC_H)
_conv_to_node = _make_conv(ACC_N, ACC_N)


def _ln(x, g, b):
    mu = jnp.mean(x, axis=-1, keepdims=True)
    var = jnp.mean((x - mu) ** 2, axis=-1, keepdims=True)
    return (x - mu) / jnp.sqrt(var + EPS) * g + b


def _k0n_body(deg_ref, h_ref, cn_ref, hpre_ref):
    d = deg_ref[0] + deg_ref[1]
    cb = jax.lax.rsqrt(jnp.maximum(d, 1.0))
    cn_ref[...] = cb
    hpre_ref[...] = h_ref[...] * cb


_BN = 400


def _k0n(deg_n_p, h):
    grid = NN // _BN
    return pl.pallas_call(
        _k0n_body,
        grid=(grid,),
        in_specs=[
            pl.BlockSpec((2, _BN, D), lambda i: (0, i, 0)),
            pl.BlockSpec((_BN, D), lambda i: (i, 0)),
        ],
        out_specs=[
            pl.BlockSpec((_BN, D), lambda i: (i, 0)),
            pl.BlockSpec((_BN, D), lambda i: (i, 0)),
        ],
        out_shape=[
            jax.ShapeDtypeStruct((NN, D), jnp.float32),
            jax.ShapeDtypeStruct((NN, D), jnp.float32),
        ],
    )(deg_n_p, h)


def _k0h_body(deg_ref, ch_ref):
    d = deg_ref[0] + deg_ref[1]
    ch_ref[...] = jax.lax.rsqrt(jnp.maximum(d, 1.0))


def _k0h(deg_h_p):
    return pl.pallas_call(
        _k0h_body,
        grid=(NH // 256,),
        in_specs=[pl.BlockSpec((2, 256, D), lambda i: (0, i, 0))],
        out_specs=pl.BlockSpec((256, D), lambda i: (i, 0)),
        out_shape=jax.ShapeDtypeStruct((NH, D), jnp.float32),
    )(deg_h_p)


def _k1_body(u_ref, ch_ref, w_ref, b_ref, out_ref):
    ch = ch_ref[...]
    u = (u_ref[0] + u_ref[1]) * ch
    h1 = jnp.dot(u, w_ref[...], preferred_element_type=jnp.float32) + b_ref[...]
    out_ref[...] = h1 * ch


def _k1(u_p, ch_b, w, b):
    return pl.pallas_call(
        _k1_body,
        grid=(1,),
        in_specs=[
            pl.BlockSpec((2, NH, D), lambda i: (0, 0, 0)),
            pl.BlockSpec((NH, D), lambda i: (0, 0)),
            pl.BlockSpec((D, D), lambda i: (0, 0)),
            pl.BlockSpec((1, D), lambda i: (0, 0)),
        ],
        out_specs=pl.BlockSpec((NH, D), lambda i: (0, 0)),
        out_shape=jax.ShapeDtypeStruct((NH, D), jnp.float32),
    )(u_p, ch_b, w, b)


def _k2_body(final, v_ref, cn_ref, h_ref, w2_ref, b2_ref, g1_ref, e1_ref,
             wf1_ref, bf1_ref, wf2_ref, bf2_ref, g2_ref, e2_ref,
             gf_ref, ef_ref, wo_ref, bo_ref, out1_ref, out2_ref):
    cn = cn_ref[...]
    hn = jnp.dot((v_ref[0] + v_ref[1]) * cn, w2_ref[...],
                 preferred_element_type=jnp.float32) + b2_ref[...]
    t = _ln(h_ref[...] + hn, g1_ref[...], e1_ref[...])
    f = jnp.maximum(
        jnp.dot(t, wf1_ref[...], preferred_element_type=jnp.float32)
        + bf1_ref[...], 0.0)
    f = jnp.dot(f, wf2_ref[...], preferred_element_type=jnp.float32) + bf2_ref[...]
    h2 = _ln(t + f, g2_ref[...], e2_ref[...])
    if final:
        hf = _ln(h2, gf_ref[...], ef_ref[...])
        out1_ref[...] = jnp.dot(hf, wo_ref[...],
                                preferred_element_type=jnp.float32) + bo_ref[...]
        out2_ref[...] = h2
    else:
        out1_ref[...] = h2
        out2_ref[...] = h2 * cn


def _k2(final, v_p, cn_b, h, w2, b2, g1, e1, wf1, bf1, wf2, bf2, g2, e2,
        gf, ef, wo, bo):
    grid = NN // _BN
    full = lambda shape: pl.BlockSpec(shape, lambda i: tuple(0 for _ in shape))
    return pl.pallas_call(
        functools.partial(_k2_body, final),
        grid=(grid,),
        in_specs=[
            pl.BlockSpec((2, _BN, D), lambda i: (0, i, 0)),
            pl.BlockSpec((_BN, D), lambda i: (i, 0)),
            pl.BlockSpec((_BN, D), lambda i: (i, 0)),
            full((D, D)), full((1, D)), full((1, D)), full((1, D)),
            full((D, FF)), full((1, FF)), full((FF, D)), full((1, D)),
            full((1, D)), full((1, D)),
            full((1, D)), full((1, D)), full((D, D)), full((1, D)),
        ],
        out_specs=[
            pl.BlockSpec((_BN, D), lambda i: (i, 0)),
            pl.BlockSpec((_BN, D), lambda i: (i, 0)),
        ],
        out_shape=[
            jax.ShapeDtypeStruct((NN, D), jnp.float32),
            jax.ShapeDtypeStruct((NN, D), jnp.float32),
        ],
    )(v_p, cn_b, h, w2, b2, g1, e1, wf1, bf1, wf2, bf2, g2, e2, gf, ef, wo, bo)


def kernel(node_features, src_idx, dst_idx, params):
    src = src_idx.astype(jnp.int32)
    dst = dst_idx.astype(jnp.int32)
    padlen = NE_PAD - NE

    def _prep(a, fill):
        return jnp.concatenate(
            [a, jnp.full((padlen,), fill, jnp.int32)]).reshape(NW, CH, K)

    src_g = _prep(src, 0)
    src_s = _prep(src, NN)
    dst_g = _prep(dst, 0)
    dst_s = _prep(dst, NH)

    zeros_rows = jnp.zeros((K, D), jnp.float32)
    ones_rows = jnp.ones((K, D), jnp.float32)

    deg_n_p = _deg_n(src_s, zeros_rows, ones_rows)
    deg_h_p = _deg_h(dst_s, zeros_rows, ones_rows)
    cn_b, h_pre = _k0n(deg_n_p, node_features)
    ch_b = _k0h(deg_h_p)

    h = node_features
    out = None
    rb = lambda x: x.reshape(1, -1)
    for l, p in enumerate(params['layers']):
        final = l == len(params['layers']) - 1
        u_p = _conv_to_hyper(h_pre, src_g, dst_s, zeros_rows)
        u2 = _k1(u_p, ch_b, p['W_n2h'], rb(p['b_n2h']))
        v_p = _conv_to_node(u2, dst_g, src_s, zeros_rows)
        out1, out2 = _k2(final, v_p, cn_b, h,
                         p['W_h2n'], rb(p['b_h2n']),
                         rb(p['ln1_g']), rb(p['ln1_b']),
                         p['W_ff1'], rb(p['b_ff1']),
                         p['W_ff2'], rb(p['b_ff2']),
                         rb(p['ln2_g']), rb(p['ln2_b']),
                         rb(params['ln_f_g']), rb(params['ln_f_b']),
                         params['W_out'], rb(params['b_out']))
        if final:
            out = out1
        else:
            h, h_pre = out1, out2
    return out


# speedup vs baseline: 2.6234x; 1.0978x over previous
import functools

import jax
import jax.numpy as jnp
from jax import lax
from jax.experimental import pallas as pl
from jax.experimental.pallas import tpu as pltpu
from jax.experimental.pallas import tpu_sc as plsc

NN = 10000
NH = 2048
NE = 320000
D = 128
FF = 4 * D
EPS = 1e-5

NW = 32
K = 128
EP = 10240
CH = EP // K
NE_PAD = NW * EP
ACC_N = 10240
ACC_H = 2560

_MESH = plsc.VectorSubcoreMesh(core_axis_name="c", subcore_axis_name="s",
                               num_cores=2, num_subcores=16)


def _zero_rows(zeros_hbm, acc, base, nrows, width):
    off = 0
    while off < nrows:
        step = min(K, nrows - off)
        pltpu.sync_copy(zeros_hbm.at[pl.ds(0, step)],
                        acc.at[pl.ds(base + off, step)])
        off += step


def _make_deg(n_out, acc_rows):
    @functools.partial(
        pl.kernel,
        mesh=_MESH,
        out_type=jax.ShapeDtypeStruct((2, n_out, D), jnp.float32),
        scratch_types=[
            pltpu.VMEM((CH, K), jnp.int32),
            pltpu.VMEM((K, D), jnp.float32),
            pltpu.VMEM_SHARED((acc_rows, D), jnp.float32),
            pltpu.SemaphoreType.DMA,
        ],
    )
    def deg(sidx_hbm, zeros_hbm, ones_hbm, out_hbm, sidx_v, ones_v, acc, sem):
        c = lax.axis_index("c")
        s = lax.axis_index("s")
        wid = s * 2 + c
        _zero_rows(zeros_hbm, acc, s * (acc_rows // 16), acc_rows // 16, D)
        pltpu.sync_copy(ones_hbm, ones_v)
        pltpu.sync_copy(sidx_hbm.at[wid], sidx_v)
        plsc.subcore_barrier()

        def body(ci, carry):
            pltpu.async_copy(ones_v, acc.at[sidx_v.at[ci]], sem, add=True)
            return carry

        lax.fori_loop(0, CH, body, 0)

        def drain(ci, carry):
            pltpu.make_async_copy(ones_v, acc.at[sidx_v.at[ci]], sem).wait()
            return carry

        lax.fori_loop(0, CH, drain, 0)
        plsc.subcore_barrier()
        rw = n_out // 16
        pltpu.sync_copy(acc.at[pl.ds(s * rw, rw)],
                        out_hbm.at[c, pl.ds(s * rw, rw)])

    return deg


_deg_n = _make_deg(ACC_N, ACC_N)
_deg_h = _make_deg(NH, ACC_H)


def _make_conv(n_out, acc_rows):
    @functools.partial(
        pl.kernel,
        mesh=_MESH,
        out_type=jax.ShapeDtypeStruct((2, n_out, D), jnp.float32),
        scratch_types=[
            pltpu.VMEM((CH // 2, K), jnp.int32),
            pltpu.VMEM((CH // 2, K), jnp.int32),
            pltpu.VMEM((K, D), jnp.float32),
            pltpu.VMEM((K, D), jnp.float32),
            pltpu.VMEM_SHARED((acc_rows, D), jnp.float32),
            pltpu.SemaphoreType.DMA,
            pltpu.SemaphoreType.DMA,
        ],
    )
    def conv(x_hbm, gidx_hbm, sidx_hbm, zeros_hbm,
             out_hbm, gidx_v, sidx_v, rows0, rows1, acc, gsem0, gsem1):
        c = lax.axis_index("c")
        s = lax.axis_index("s")
        wid = s * 2 + c
        rows = (rows0, rows1)
        gsem = (gsem0, gsem1)
        ch2 = CH // 2
        _zero_rows(zeros_hbm, acc, s * (acc_rows // 16), acc_rows // 16, D)
        plsc.subcore_barrier()

        for half in (0, 1):
            pltpu.sync_copy(gidx_hbm.at[wid, pl.ds(half * ch2, ch2)], gidx_v)
            pltpu.sync_copy(sidx_hbm.at[wid, pl.ds(half * ch2, ch2)], sidx_v)
            pltpu.async_copy(x_hbm.at[gidx_v.at[0]], rows0, gsem0)

            def outer(o, carry):
                for b in (0, 1):
                    ci = o * 2 + b
                    nxt = ci + 1

                    @pl.when(nxt < ch2)
                    def _():
                        pltpu.async_copy(x_hbm.at[gidx_v.at[nxt]],
                                         rows[1 - b], gsem[1 - b])

                    pltpu.make_async_copy(x_hbm.at[gidx_v.at[ci]],
                                          rows[b], gsem[b]).wait()
                    pltpu.sync_copy(rows[b], acc.at[sidx_v.at[ci]], add=True)
                return carry

            lax.fori_loop(0, ch2 // 2, outer, 0)
        plsc.subcore_barrier()
        rw = n_out // 16
        pltpu.sync_copy(acc.at[pl.ds(s * rw, rw)],
                        out_hbm.at[c, pl.ds(s * rw, rw)])

    return conv


_conv_to_hyper = _make_conv(NH, ACC_H)
_conv_to_node = _make_conv(ACC_N, ACC_N)


def _ln(x, g, b):
    mu = jnp.mean(x, axis=-1, keepdims=True)
    var = jnp.mean((x - mu) ** 2, axis=-1, keepdims=True)
    return (x - mu) / jnp.sqrt(var + EPS) * g + b


def _k0n_body(deg_ref, h_ref, cn_ref, hpre_ref):
    d = deg_ref[0] + deg_ref[1]
    cb = jax.lax.rsqrt(jnp.maximum(d, 1.0))
    cn_ref[...] = cb
    hpre_ref[...] = h_ref[...] * cb


_BN = 400


def _k0n(deg_n_p, h):
    grid = NN // _BN
    return pl.pallas_call(
        _k0n_body,
        grid=(grid,),
        in_specs=[
            pl.BlockSpec((2, _BN, D), lambda i: (0, i, 0)),
            pl.BlockSpec((_BN, D), lambda i: (i, 0)),
        ],
        out_specs=[
            pl.BlockSpec((_BN, D), lambda i: (i, 0)),
            pl.BlockSpec((_BN, D), lambda i: (i, 0)),
        ],
        out_shape=[
            jax.ShapeDtypeStruct((NN, D), jnp.float32),
            jax.ShapeDtypeStruct((NN, D), jnp.float32),
        ],
    )(deg_n_p, h)


def _k0h_body(deg_ref, ch_ref):
    d = deg_ref[0] + deg_ref[1]
    ch_ref[...] = jax.lax.rsqrt(jnp.maximum(d, 1.0))


def _k0h(deg_h_p):
    return pl.pallas_call(
        _k0h_body,
        grid=(NH // 256,),
        in_specs=[pl.BlockSpec((2, 256, D), lambda i: (0, i, 0))],
        out_specs=pl.BlockSpec((256, D), lambda i: (i, 0)),
        out_shape=jax.ShapeDtypeStruct((NH, D), jnp.float32),
    )(deg_h_p)


def _k1_body(u_ref, ch_ref, w_ref, b_ref, out_ref):
    ch = ch_ref[...]
    u = (u_ref[0] + u_ref[1]) * ch
    h1 = jnp.dot(u, w_ref[...], preferred_element_type=jnp.float32) + b_ref[...]
    out_ref[...] = h1 * ch


def _k1(u_p, ch_b, w, b):
    return pl.pallas_call(
        _k1_body,
        grid=(1,),
        in_specs=[
            pl.BlockSpec((2, NH, D), lambda i: (0, 0, 0)),
            pl.BlockSpec((NH, D), lambda i: (0, 0)),
            pl.BlockSpec((D, D), lambda i: (0, 0)),
            pl.BlockSpec((1, D), lambda i: (0, 0)),
        ],
        out_specs=pl.BlockSpec((NH, D), lambda i: (0, 0)),
        out_shape=jax.ShapeDtypeStruct((NH, D), jnp.float32),
    )(u_p, ch_b, w, b)


def _k2_body(final, v_ref, cn_ref, h_ref, w2_ref, b2_ref, g1_ref, e1_ref,
             wf1_ref, bf1_ref, wf2_ref, bf2_ref, g2_ref, e2_ref,
             gf_ref, ef_ref, wo_ref, bo_ref, out1_ref, out2_ref):
    cn = cn_ref[...]
    hn = jnp.dot((v_ref[0] + v_ref[1]) * cn, w2_ref[...],
                 preferred_element_type=jnp.float32) + b2_ref[...]
    t = _ln(h_ref[...] + hn, g1_ref[...], e1_ref[...])
    f = jnp.maximum(
        jnp.dot(t, wf1_ref[...], preferred_element_type=jnp.float32)
        + bf1_ref[...], 0.0)
    f = jnp.dot(f, wf2_ref[...], preferred_element_type=jnp.float32) + bf2_ref[...]
    h2 = _ln(t + f, g2_ref[...], e2_ref[...])
    if final:
        hf = _ln(h2, gf_ref[...], ef_ref[...])
        out1_ref[...] = jnp.dot(hf, wo_ref[...],
                                preferred_element_type=jnp.float32) + bo_ref[...]
        out2_ref[...] = h2
    else:
        out1_ref[...] = h2
        out2_ref[...] = h2 * cn


def _k2(final, v_p, cn_b, h, w2, b2, g1, e1, wf1, bf1, wf2, bf2, g2, e2,
        gf, ef, wo, bo):
    grid = NN // _BN
    full = lambda shape: pl.BlockSpec(shape, lambda i: tuple(0 for _ in shape))
    return pl.pallas_call(
        functools.partial(_k2_body, final),
        grid=(grid,),
        in_specs=[
            pl.BlockSpec((2, _BN, D), lambda i: (0, i, 0)),
            pl.BlockSpec((_BN, D), lambda i: (i, 0)),
            pl.BlockSpec((_BN, D), lambda i: (i, 0)),
            full((D, D)), full((1, D)), full((1, D)), full((1, D)),
            full((D, FF)), full((1, FF)), full((FF, D)), full((1, D)),
            full((1, D)), full((1, D)),
            full((1, D)), full((1, D)), full((D, D)), full((1, D)),
        ],
        out_specs=[
            pl.BlockSpec((_BN, D), lambda i: (i, 0)),
            pl.BlockSpec((_BN, D), lambda i: (i, 0)),
        ],
        out_shape=[
            jax.ShapeDtypeStruct((NN, D), jnp.float32),
            jax.ShapeDtypeStruct((NN, D), jnp.float32),
        ],
    )(v_p, cn_b, h, w2, b2, g1, e1, wf1, bf1, wf2, bf2, g2, e2, gf, ef, wo, bo)


def kernel(node_features, src_idx, dst_idx, params):
    src = src_idx.astype(jnp.int32)
    dst = dst_idx.astype(jnp.int32)
    padlen = NE_PAD - NE

    def _prep(a, fill):
        return jnp.concatenate(
            [a, jnp.full((padlen,), fill, jnp.int32)]).reshape(NW, CH, K)

    src_g = _prep(src, 0)
    src_s = _prep(src, NN)
    dst_g = _prep(dst, 0)
    dst_s = _prep(dst, NH)

    zeros_rows = jnp.zeros((K, D), jnp.float32)
    ones_rows = jnp.ones((K, D), jnp.float32)

    deg_n_p = _deg_n(src_s, zeros_rows, ones_rows)
    deg_h_p = _deg_h(dst_s, zeros_rows, ones_rows)
    cn_b, h_pre = _k0n(deg_n_p, node_features)
    ch_b = _k0h(deg_h_p)

    h = node_features
    out = None
    rb = lambda x: x.reshape(1, -1)
    for l, p in enumerate(params['layers']):
        final = l == len(params['layers']) - 1
        u_p = _conv_to_hyper(h_pre, src_g, dst_s, zeros_rows)
        u2 = _k1(u_p, ch_b, p['W_n2h'], rb(p['b_n2h']))
        v_p = _conv_to_node(u2, dst_g, src_s, zeros_rows)
        out1, out2 = _k2(final, v_p, cn_b, h,
                         p['W_h2n'], rb(p['b_h2n']),
                         rb(p['ln1_g']), rb(p['ln1_b']),
                         p['W_ff1'], rb(p['b_ff1']),
                         p['W_ff2'], rb(p['b_ff2']),
                         rb(p['ln2_g']), rb(p['ln2_b']),
                         rb(params['ln_f_g']), rb(params['ln_f_b']),
                         params['W_out'], rb(params['b_out']))
        if final:
            out = out1
        else:
            h, h_pre = out1, out2
    return out


# speedup vs baseline: 3.0789x; 1.1736x over previous
import functools

import jax
import jax.numpy as jnp
from jax import lax
from jax.experimental import pallas as pl
from jax.experimental.pallas import tpu as pltpu
from jax.experimental.pallas import tpu_sc as plsc

NN = 10000
NH = 2048
NE = 320000
D = 128
FF = 4 * D
EPS = 1e-5

NW = 32
K = 128
EP = 10240
CH = EP // K
NE_PAD = NW * EP
ACC_N = 10240
ACC_H = 2560

_MESH = plsc.VectorSubcoreMesh(core_axis_name="c", subcore_axis_name="s",
                               num_cores=2, num_subcores=16)


def _zero_rows(zeros_hbm, acc, base, nrows, width):
    off = 0
    while off < nrows:
        step = min(K, nrows - off)
        pltpu.sync_copy(zeros_hbm.at[pl.ds(0, step)],
                        acc.at[pl.ds(base + off, step)])
        off += step


def _make_deg(n_out, acc_rows):
    @functools.partial(
        pl.kernel,
        mesh=_MESH,
        out_type=jax.ShapeDtypeStruct((2, n_out, D), jnp.float32),
        scratch_types=[
            pltpu.VMEM((CH, K), jnp.int32),
            pltpu.VMEM((K, D), jnp.float32),
            pltpu.VMEM_SHARED((acc_rows, D), jnp.float32),
            pltpu.SemaphoreType.DMA,
        ],
    )
    def deg(sidx_hbm, zeros_hbm, ones_hbm, out_hbm, sidx_v, ones_v, acc, sem):
        c = lax.axis_index("c")
        s = lax.axis_index("s")
        wid = s * 2 + c
        _zero_rows(zeros_hbm, acc, s * (acc_rows // 16), acc_rows // 16, D)
        pltpu.sync_copy(ones_hbm, ones_v)
        pltpu.sync_copy(sidx_hbm.at[wid], sidx_v)
        plsc.subcore_barrier()

        def body(ci, carry):
            pltpu.async_copy(ones_v, acc.at[sidx_v.at[ci]], sem, add=True)
            return carry

        lax.fori_loop(0, CH, body, 0)

        def drain(ci, carry):
            pltpu.make_async_copy(ones_v, acc.at[sidx_v.at[ci]], sem).wait()
            return carry

        lax.fori_loop(0, CH, drain, 0)
        plsc.subcore_barrier()
        rw = n_out // 16
        pltpu.sync_copy(acc.at[pl.ds(s * rw, rw)],
                        out_hbm.at[c, pl.ds(s * rw, rw)])

    return deg


_deg_n = _make_deg(ACC_N, ACC_N)
_deg_h = _make_deg(NH, ACC_H)


def _make_conv(n_out, acc_rows, k, nbuf, n_stages):
    ch = EP // k
    ch2 = ch // n_stages
    lead = nbuf - 1

    @functools.partial(
        pl.kernel,
        mesh=_MESH,
        out_type=jax.ShapeDtypeStruct((2, n_out, D), jnp.float32),
        scratch_types=(
            [pltpu.VMEM((ch2, k), jnp.int32)] * 2
            + [pltpu.VMEM((k, D), jnp.float32)] * nbuf
            + [pltpu.VMEM_SHARED((acc_rows, D), jnp.float32)]
            + [pltpu.SemaphoreType.DMA] * (2 * nbuf)
        ),
    )
    def conv(x_hbm, gidx_hbm, sidx_hbm, zeros_hbm, out_hbm, *scr):
        gidx_v, sidx_v = scr[0], scr[1]
        rows = scr[2:2 + nbuf]
        acc = scr[2 + nbuf]
        gsem = scr[3 + nbuf:3 + 2 * nbuf]
        ssem = scr[3 + 2 * nbuf:3 + 3 * nbuf]
        c = lax.axis_index("c")
        s = lax.axis_index("s")
        wid = s * 2 + c
        _zero_rows(zeros_hbm, acc, s * (acc_rows // 16), acc_rows // 16, D)
        plsc.subcore_barrier()

        for half in range(n_stages):
            pltpu.sync_copy(gidx_hbm.at[wid, pl.ds(half * ch2, ch2)], gidx_v)
            pltpu.sync_copy(sidx_hbm.at[wid, pl.ds(half * ch2, ch2)], sidx_v)
            for j in range(lead):
                pltpu.async_copy(x_hbm.at[gidx_v.at[j]], rows[j], gsem[j])

            def outer(o, carry):
                for b in range(nbuf):
                    ci = o * nbuf + b
                    nxt = ci + lead
                    bn = (b + lead) % nbuf

                    @pl.when(nxt < ch2)
                    def _():
                        @pl.when(nxt >= nbuf)
                        def _():
                            pltpu.make_async_copy(
                                rows[bn], acc.at[sidx_v.at[0]],
                                ssem[bn]).wait()

                        pltpu.async_copy(x_hbm.at[gidx_v.at[nxt]],
                                         rows[bn], gsem[bn])

                    pltpu.make_async_copy(x_hbm.at[gidx_v.at[ci]],
                                          rows[b], gsem[b]).wait()
                    pltpu.async_copy(rows[b], acc.at[sidx_v.at[ci]],
                                     ssem[b], add=True)
                return carry

            lax.fori_loop(0, ch2 // nbuf, outer, 0)
            for j in range(nbuf):
                pltpu.make_async_copy(rows[j], acc.at[sidx_v.at[0]],
                                      ssem[j]).wait()
        plsc.subcore_barrier()
        rw = n_out // 16
        pltpu.sync_copy(acc.at[pl.ds(s * rw, rw)],
                        out_hbm.at[c, pl.ds(s * rw, rw)])

    return conv


K_N = 64
_conv_to_hyper = _make_conv(NH, ACC_H, K, 4, 2)
_conv_to_node = _make_conv(ACC_N, ACC_N, K_N, 4, 4)


def _ln(x, g, b):
    mu = jnp.mean(x, axis=-1, keepdims=True)
    var = jnp.mean((x - mu) ** 2, axis=-1, keepdims=True)
    return (x - mu) / jnp.sqrt(var + EPS) * g + b


def _k0n_body(deg_ref, h_ref, cn_ref, hpre_ref):
    d = deg_ref[0] + deg_ref[1]
    cb = jax.lax.rsqrt(jnp.maximum(d, 1.0))
    cn_ref[...] = cb
    hpre_ref[...] = h_ref[...] * cb


_BN = 400


def _k0n(deg_n_p, h):
    grid = NN // _BN
    return pl.pallas_call(
        _k0n_body,
        grid=(grid,),
        in_specs=[
            pl.BlockSpec((2, _BN, D), lambda i: (0, i, 0)),
            pl.BlockSpec((_BN, D), lambda i: (i, 0)),
        ],
        out_specs=[
            pl.BlockSpec((_BN, D), lambda i: (i, 0)),
            pl.BlockSpec((_BN, D), lambda i: (i, 0)),
        ],
        out_shape=[
            jax.ShapeDtypeStruct((NN, D), jnp.float32),
            jax.ShapeDtypeStruct((NN, D), jnp.float32),
        ],
    )(deg_n_p, h)


def _k0h_body(deg_ref, ch_ref):
    d = deg_ref[0] + deg_ref[1]
    ch_ref[...] = jax.lax.rsqrt(jnp.maximum(d, 1.0))


def _k0h(deg_h_p):
    return pl.pallas_call(
        _k0h_body,
        grid=(NH // 256,),
        in_specs=[pl.BlockSpec((2, 256, D), lambda i: (0, i, 0))],
        out_specs=pl.BlockSpec((256, D), lambda i: (i, 0)),
        out_shape=jax.ShapeDtypeStruct((NH, D), jnp.float32),
    )(deg_h_p)


def _k1_body(u_ref, ch_ref, w_ref, b_ref, out_ref):
    ch = ch_ref[...]
    u = (u_ref[0] + u_ref[1]) * ch
    h1 = jnp.dot(u, w_ref[...], preferred_element_type=jnp.float32) + b_ref[...]
    out_ref[...] = h1 * ch


def _k1(u_p, ch_b, w, b):
    return pl.pallas_call(
        _k1_body,
        grid=(1,),
        in_specs=[
            pl.BlockSpec((2, NH, D), lambda i: (0, 0, 0)),
            pl.BlockSpec((NH, D), lambda i: (0, 0)),
            pl.BlockSpec((D, D), lambda i: (0, 0)),
            pl.BlockSpec((1, D), lambda i: (0, 0)),
        ],
        out_specs=pl.BlockSpec((NH, D), lambda i: (0, 0)),
        out_shape=jax.ShapeDtypeStruct((NH, D), jnp.float32),
    )(u_p, ch_b, w, b)


def _k2_body(final, v_ref, cn_ref, h_ref, w2_ref, b2_ref, g1_ref, e1_ref,
             wf1_ref, bf1_ref, wf2_ref, bf2_ref, g2_ref, e2_ref,
             gf_ref, ef_ref, wo_ref, bo_ref, out1_ref, out2_ref):
    cn = cn_ref[...]
    hn = jnp.dot((v_ref[0] + v_ref[1]) * cn, w2_ref[...],
                 preferred_element_type=jnp.float32) + b2_ref[...]
    t = _ln(h_ref[...] + hn, g1_ref[...], e1_ref[...])
    f = jnp.maximum(
        jnp.dot(t, wf1_ref[...], preferred_element_type=jnp.float32)
        + bf1_ref[...], 0.0)
    f = jnp.dot(f, wf2_ref[...], preferred_element_type=jnp.float32) + bf2_ref[...]
    h2 = _ln(t + f, g2_ref[...], e2_ref[...])
    if final:
        hf = _ln(h2, gf_ref[...], ef_ref[...])
        out1_ref[...] = jnp.dot(hf, wo_ref[...],
                                preferred_element_type=jnp.float32) + bo_ref[...]
        out2_ref[...] = h2
    else:
        out1_ref[...] = h2
        out2_ref[...] = h2 * cn


def _k2(final, v_p, cn_b, h, w2, b2, g1, e1, wf1, bf1, wf2, bf2, g2, e2,
        gf, ef, wo, bo):
    grid = NN // _BN
    full = lambda shape: pl.BlockSpec(shape, lambda i: tuple(0 for _ in shape))
    return pl.pallas_call(
        functools.partial(_k2_body, final),
        grid=(grid,),
        in_specs=[
            pl.BlockSpec((2, _BN, D), lambda i: (0, i, 0)),
            pl.BlockSpec((_BN, D), lambda i: (i, 0)),
            pl.BlockSpec((_BN, D), lambda i: (i, 0)),
            full((D, D)), full((1, D)), full((1, D)), full((1, D)),
            full((D, FF)), full((1, FF)), full((FF, D)), full((1, D)),
            full((1, D)), full((1, D)),
            full((1, D)), full((1, D)), full((D, D)), full((1, D)),
        ],
        out_specs=[
            pl.BlockSpec((_BN, D), lambda i: (i, 0)),
            pl.BlockSpec((_BN, D), lambda i: (i, 0)),
        ],
        out_shape=[
            jax.ShapeDtypeStruct((NN, D), jnp.float32),
            jax.ShapeDtypeStruct((NN, D), jnp.float32),
        ],
    )(v_p, cn_b, h, w2, b2, g1, e1, wf1, bf1, wf2, bf2, g2, e2, gf, ef, wo, bo)


def kernel(node_features, src_idx, dst_idx, params):
    src = src_idx.astype(jnp.int32)
    dst = dst_idx.astype(jnp.int32)
    padlen = NE_PAD - NE

    def _prep(a, fill, k):
        return jnp.concatenate(
            [a, jnp.full((padlen,), fill, jnp.int32)]).reshape(NW, EP // k, k)

    src_g = _prep(src, 0, K)
    src_s128 = _prep(src, NN, K)
    dst_s = _prep(dst, NH, K)
    dst_g = _prep(dst, 0, K_N)
    src_s = _prep(src, NN, K_N)

    zeros_rows = jnp.zeros((K, D), jnp.float32)
    ones_rows = jnp.ones((K, D), jnp.float32)

    deg_n_p = _deg_n(src_s128, zeros_rows, ones_rows)
    deg_h_p = _deg_h(dst_s, zeros_rows, ones_rows)
    cn_b, h_pre = _k0n(deg_n_p, node_features)
    ch_b = _k0h(deg_h_p)

    h = node_features
    out = None
    rb = lambda x: x.reshape(1, -1)
    for l, p in enumerate(params['layers']):
        final = l == len(params['layers']) - 1
        u_p = _conv_to_hyper(h_pre, src_g, dst_s, zeros_rows)
        u2 = _k1(u_p, ch_b, p['W_n2h'], rb(p['b_n2h']))
        v_p = _conv_to_node(u2, dst_g, src_s, zeros_rows)
        out1, out2 = _k2(final, v_p, cn_b, h,
                         p['W_h2n'], rb(p['b_h2n']),
                         rb(p['ln1_g']), rb(p['ln1_b']),
                         p['W_ff1'], rb(p['b_ff1']),
                         p['W_ff2'], rb(p['b_ff2']),
                         rb(p['ln2_g']), rb(p['ln2_b']),
                         rb(params['ln_f_g']), rb(params['ln_f_b']),
                         params['W_out'], rb(params['b_out']))
        if final:
            out = out1
        else:
            h, h_pre = out1, out2
    return out


# speedup vs baseline: 4.4418x; 1.4427x over previous
import functools

import jax
import jax.numpy as jnp
from jax import lax
from jax.experimental import pallas as pl
from jax.experimental.pallas import tpu as pltpu
from jax.experimental.pallas import tpu_sc as plsc

NN = 10000
NH = 2048
NE = 320000
D = 128
FF = 4 * D
EPS = 1e-5

NW = 32
K = 128
EP = 10240
CH = EP // K
NE_PAD = NW * EP
ACC_N = 10240
ACC_H = 2560

_MESH = plsc.VectorSubcoreMesh(core_axis_name="c", subcore_axis_name="s",
                               num_cores=2, num_subcores=16)


def _zero_rows(zeros_hbm, acc, base, nrows, width):
    off = 0
    while off < nrows:
        step = min(K, nrows - off)
        pltpu.sync_copy(zeros_hbm.at[pl.ds(0, step)],
                        acc.at[pl.ds(base + off, step)])
        off += step


def _make_deg(n_out, acc_rows):
    @functools.partial(
        pl.kernel,
        mesh=_MESH,
        out_type=jax.ShapeDtypeStruct((2, n_out, D), jnp.float32),
        scratch_types=[
            pltpu.VMEM((CH, K), jnp.int32),
            pltpu.VMEM((K, D), jnp.float32),
            pltpu.VMEM_SHARED((acc_rows, D), jnp.float32),
            pltpu.SemaphoreType.DMA,
        ],
    )
    def deg(sidx_hbm, zeros_hbm, ones_hbm, out_hbm, sidx_v, ones_v, acc, sem):
        c = lax.axis_index("c")
        s = lax.axis_index("s")
        wid = s * 2 + c
        _zero_rows(zeros_hbm, acc, s * (acc_rows // 16), acc_rows // 16, D)
        pltpu.sync_copy(ones_hbm, ones_v)
        pltpu.sync_copy(sidx_hbm.at[wid], sidx_v)
        plsc.subcore_barrier()

        def body(ci, carry):
            pltpu.async_copy(ones_v, acc.at[sidx_v.at[ci]], sem, add=True)
            return carry

        lax.fori_loop(0, CH, body, 0)

        def drain(ci, carry):
            pltpu.make_async_copy(ones_v, acc.at[sidx_v.at[ci]], sem).wait()
            return carry

        lax.fori_loop(0, CH, drain, 0)
        plsc.subcore_barrier()
        rw = n_out // 16
        pltpu.sync_copy(acc.at[pl.ds(s * rw, rw)],
                        out_hbm.at[c, pl.ds(s * rw, rw)])

    return deg


_deg_n = _make_deg(ACC_N, ACC_N)
_deg_h = _make_deg(NH, ACC_H)


def _make_conv(n_out, acc_rows, k, nbuf, n_stages, table_rows=0):
    ch = EP // k
    ch2 = ch // n_stages
    lead = nbuf - 1

    @functools.partial(
        pl.kernel,
        mesh=_MESH,
        out_type=jax.ShapeDtypeStruct((2, n_out, D), jnp.float32),
        scratch_types=(
            [pltpu.VMEM((ch2, k), jnp.int32)] * 2
            + [pltpu.VMEM((k, D), jnp.float32)] * nbuf
            + [pltpu.VMEM_SHARED((acc_rows, D), jnp.float32)]
            + ([pltpu.VMEM_SHARED((table_rows, D), jnp.float32)]
               if table_rows else [])
            + [pltpu.SemaphoreType.DMA] * (2 * nbuf)
        ),
    )
    def conv(x_hbm, gidx_hbm, sidx_hbm, zeros_hbm, out_hbm, *scr):
        gidx_v, sidx_v = scr[0], scr[1]
        rows = scr[2:2 + nbuf]
        acc = scr[2 + nbuf]
        off = 3 + nbuf + (1 if table_rows else 0)
        gsem = scr[off:off + nbuf]
        ssem = scr[off + nbuf:off + 2 * nbuf]
        c = lax.axis_index("c")
        s = lax.axis_index("s")
        wid = s * 2 + c
        _zero_rows(zeros_hbm, acc, s * (acc_rows // 16), acc_rows // 16, D)
        if table_rows:
            table = scr[2 + nbuf + 1]
            tr = table_rows // 16
            pltpu.sync_copy(x_hbm.at[pl.ds(s * tr, tr)],
                            table.at[pl.ds(s * tr, tr)])
            x_src = table
        else:
            x_src = x_hbm
        plsc.subcore_barrier()

        for half in range(n_stages):
            pltpu.sync_copy(gidx_hbm.at[wid, pl.ds(half * ch2, ch2)], gidx_v)
            pltpu.sync_copy(sidx_hbm.at[wid, pl.ds(half * ch2, ch2)], sidx_v)
            for j in range(lead):
                pltpu.async_copy(x_src.at[gidx_v.at[j]], rows[j], gsem[j])

            def outer(o, carry):
                for b in range(nbuf):
                    ci = o * nbuf + b
                    nxt = ci + lead
                    bn = (b + lead) % nbuf

                    @pl.when(nxt < ch2)
                    def _():
                        @pl.when(nxt >= nbuf)
                        def _():
                            pltpu.make_async_copy(
                                rows[bn], acc.at[sidx_v.at[0]],
                                ssem[bn]).wait()

                        pltpu.async_copy(x_src.at[gidx_v.at[nxt]],
                                         rows[bn], gsem[bn])

                    pltpu.make_async_copy(x_src.at[gidx_v.at[ci]],
                                          rows[b], gsem[b]).wait()
                    pltpu.async_copy(rows[b], acc.at[sidx_v.at[ci]],
                                     ssem[b], add=True)
                return carry

            lax.fori_loop(0, ch2 // nbuf, outer, 0)
            for j in range(nbuf):
                pltpu.make_async_copy(rows[j], acc.at[sidx_v.at[0]],
                                      ssem[j]).wait()
        plsc.subcore_barrier()
        rw = n_out // 16
        pltpu.sync_copy(acc.at[pl.ds(s * rw, rw)],
                        out_hbm.at[c, pl.ds(s * rw, rw)])

    return conv


K_N = 64
_conv_to_hyper = _make_conv(NH, ACC_H, K, 4, 2)
_conv_to_node = _make_conv(ACC_N, ACC_N, K_N, 2, 4, table_rows=NH)


def _ln(x, g, b):
    mu = jnp.mean(x, axis=-1, keepdims=True)
    var = jnp.mean((x - mu) ** 2, axis=-1, keepdims=True)
    return (x - mu) / jnp.sqrt(var + EPS) * g + b


def _k0n_body(deg_ref, h_ref, cn_ref, hpre_ref):
    d = deg_ref[0] + deg_ref[1]
    cb = jax.lax.rsqrt(jnp.maximum(d, 1.0))
    cn_ref[...] = cb
    hpre_ref[...] = h_ref[...] * cb


_BN = 400


def _k0n(deg_n_p, h):
    grid = NN // _BN
    return pl.pallas_call(
        _k0n_body,
        grid=(grid,),
        in_specs=[
            pl.BlockSpec((2, _BN, D), lambda i: (0, i, 0)),
            pl.BlockSpec((_BN, D), lambda i: (i, 0)),
        ],
        out_specs=[
            pl.BlockSpec((_BN, D), lambda i: (i, 0)),
            pl.BlockSpec((_BN, D), lambda i: (i, 0)),
        ],
        out_shape=[
            jax.ShapeDtypeStruct((NN, D), jnp.float32),
            jax.ShapeDtypeStruct((NN, D), jnp.float32),
        ],
    )(deg_n_p, h)


def _k0h_body(deg_ref, ch_ref):
    d = deg_ref[0] + deg_ref[1]
    ch_ref[...] = jax.lax.rsqrt(jnp.maximum(d, 1.0))


def _k0h(deg_h_p):
    return pl.pallas_call(
        _k0h_body,
        grid=(NH // 256,),
        in_specs=[pl.BlockSpec((2, 256, D), lambda i: (0, i, 0))],
        out_specs=pl.BlockSpec((256, D), lambda i: (i, 0)),
        out_shape=jax.ShapeDtypeStruct((NH, D), jnp.float32),
    )(deg_h_p)


def _k1_body(u_ref, ch_ref, w_ref, b_ref, out_ref):
    ch = ch_ref[...]
    u = (u_ref[0] + u_ref[1]) * ch
    h1 = jnp.dot(u, w_ref[...], preferred_element_type=jnp.float32) + b_ref[...]
    out_ref[...] = h1 * ch


def _k1(u_p, ch_b, w, b):
    return pl.pallas_call(
        _k1_body,
        grid=(1,),
        in_specs=[
            pl.BlockSpec((2, NH, D), lambda i: (0, 0, 0)),
            pl.BlockSpec((NH, D), lambda i: (0, 0)),
            pl.BlockSpec((D, D), lambda i: (0, 0)),
            pl.BlockSpec((1, D), lambda i: (0, 0)),
        ],
        out_specs=pl.BlockSpec((NH, D), lambda i: (0, 0)),
        out_shape=jax.ShapeDtypeStruct((NH, D), jnp.float32),
    )(u_p, ch_b, w, b)


def _k2_body(final, v_ref, cn_ref, h_ref, w2_ref, b2_ref, g1_ref, e1_ref,
             wf1_ref, bf1_ref, wf2_ref, bf2_ref, g2_ref, e2_ref,
             gf_ref, ef_ref, wo_ref, bo_ref, out1_ref, out2_ref):
    cn = cn_ref[...]
    hn = jnp.dot((v_ref[0] + v_ref[1]) * cn, w2_ref[...],
                 preferred_element_type=jnp.float32) + b2_ref[...]
    t = _ln(h_ref[...] + hn, g1_ref[...], e1_ref[...])
    f = jnp.maximum(
        jnp.dot(t, wf1_ref[...], preferred_element_type=jnp.float32)
        + bf1_ref[...], 0.0)
    f = jnp.dot(f, wf2_ref[...], preferred_element_type=jnp.float32) + bf2_ref[...]
    h2 = _ln(t + f, g2_ref[...], e2_ref[...])
    if final:
        hf = _ln(h2, gf_ref[...], ef_ref[...])
        out1_ref[...] = jnp.dot(hf, wo_ref[...],
                                preferred_element_type=jnp.float32) + bo_ref[...]
        out2_ref[...] = h2
    else:
        out1_ref[...] = h2
        out2_ref[...] = h2 * cn


def _k2(final, v_p, cn_b, h, w2, b2, g1, e1, wf1, bf1, wf2, bf2, g2, e2,
        gf, ef, wo, bo):
    grid = NN // _BN
    full = lambda shape: pl.BlockSpec(shape, lambda i: tuple(0 for _ in shape))
    return pl.pallas_call(
        functools.partial(_k2_body, final),
        grid=(grid,),
        in_specs=[
            pl.BlockSpec((2, _BN, D), lambda i: (0, i, 0)),
            pl.BlockSpec((_BN, D), lambda i: (i, 0)),
            pl.BlockSpec((_BN, D), lambda i: (i, 0)),
            full((D, D)), full((1, D)), full((1, D)), full((1, D)),
            full((D, FF)), full((1, FF)), full((FF, D)), full((1, D)),
            full((1, D)), full((1, D)),
            full((1, D)), full((1, D)), full((D, D)), full((1, D)),
        ],
        out_specs=[
            pl.BlockSpec((_BN, D), lambda i: (i, 0)),
            pl.BlockSpec((_BN, D), lambda i: (i, 0)),
        ],
        out_shape=[
            jax.ShapeDtypeStruct((NN, D), jnp.float32),
            jax.ShapeDtypeStruct((NN, D), jnp.float32),
        ],
    )(v_p, cn_b, h, w2, b2, g1, e1, wf1, bf1, wf2, bf2, g2, e2, gf, ef, wo, bo)


def kernel(node_features, src_idx, dst_idx, params):
    src = src_idx.astype(jnp.int32)
    dst = dst_idx.astype(jnp.int32)
    padlen = NE_PAD - NE

    def _prep(a, fill, k):
        return jnp.concatenate(
            [a, jnp.full((padlen,), fill, jnp.int32)]).reshape(NW, EP // k, k)

    src_g = _prep(src, 0, K)
    src_s128 = _prep(src, NN, K)
    dst_s = _prep(dst, NH, K)
    dst_g = _prep(dst, 0, K_N)
    src_s = _prep(src, NN, K_N)

    zeros_rows = jnp.zeros((K, D), jnp.float32)
    ones_rows = jnp.ones((K, D), jnp.float32)

    deg_n_p = _deg_n(src_s128, zeros_rows, ones_rows)
    deg_h_p = _deg_h(dst_s, zeros_rows, ones_rows)
    cn_b, h_pre = _k0n(deg_n_p, node_features)
    ch_b = _k0h(deg_h_p)

    h = node_features
    out = None
    rb = lambda x: x.reshape(1, -1)
    for l, p in enumerate(params['layers']):
        final = l == len(params['layers']) - 1
        u_p = _conv_to_hyper(h_pre, src_g, dst_s, zeros_rows)
        u2 = _k1(u_p, ch_b, p['W_n2h'], rb(p['b_n2h']))
        v_p = _conv_to_node(u2, dst_g, src_s, zeros_rows)
        out1, out2 = _k2(final, v_p, cn_b, h,
                         p['W_h2n'], rb(p['b_h2n']),
                         rb(p['ln1_g']), rb(p['ln1_b']),
                         p['W_ff1'], rb(p['b_ff1']),
                         p['W_ff2'], rb(p['b_ff2']),
                         rb(p['ln2_g']), rb(p['ln2_b']),
                         rb(params['ln_f_g']), rb(params['ln_f_b']),
                         params['W_out'], rb(params['b_out']))
        if final:
            out = out1
        else:
            h, h_pre = out1, out2
    return out


# speedup vs baseline: 7.2360x; 1.6290x over previous
import functools

import jax
import jax.numpy as jnp
from jax import lax
from jax.experimental import pallas as pl
from jax.experimental.pallas import tpu as pltpu
from jax.experimental.pallas import tpu_sc as plsc

NN = 10000
NH = 2048
NE = 320000
D = 128
FF = 4 * D
EPS = 1e-5

NW = 32
K = 128
EP = 10240
CH = EP // K
NE_PAD = NW * EP
ACC_N = 10240
ACC_H = 2560

_MESH = plsc.VectorSubcoreMesh(core_axis_name="c", subcore_axis_name="s",
                               num_cores=2, num_subcores=16)


def _zero_rows(zeros_hbm, acc, base, nrows, width):
    off = 0
    while off < nrows:
        step = min(K, nrows - off)
        pltpu.sync_copy(zeros_hbm.at[pl.ds(0, step)],
                        acc.at[pl.ds(base + off, step)])
        off += step


def _make_deg(n_out, acc_rows):
    @functools.partial(
        pl.kernel,
        mesh=_MESH,
        out_type=jax.ShapeDtypeStruct((2, n_out, D), jnp.float32),
        scratch_types=[
            pltpu.VMEM((CH, K), jnp.int32),
            pltpu.VMEM((K, D), jnp.float32),
            pltpu.VMEM_SHARED((acc_rows, D), jnp.float32),
            pltpu.SemaphoreType.DMA,
        ],
    )
    def deg(sidx_hbm, zeros_hbm, ones_hbm, out_hbm, sidx_v, ones_v, acc, sem):
        c = lax.axis_index("c")
        s = lax.axis_index("s")
        wid = s * 2 + c
        _zero_rows(zeros_hbm, acc, s * (acc_rows // 16), acc_rows // 16, D)
        pltpu.sync_copy(ones_hbm, ones_v)
        pltpu.sync_copy(sidx_hbm.at[wid], sidx_v)
        plsc.subcore_barrier()

        def body(ci, carry):
            pltpu.async_copy(ones_v, acc.at[sidx_v.at[ci]], sem, add=True)
            return carry

        lax.fori_loop(0, CH, body, 0)

        def drain(ci, carry):
            pltpu.make_async_copy(ones_v, acc.at[sidx_v.at[ci]], sem).wait()
            return carry

        lax.fori_loop(0, CH, drain, 0)
        plsc.subcore_barrier()
        rw = n_out // 16
        pltpu.sync_copy(acc.at[pl.ds(s * rw, rw)],
                        out_hbm.at[c, pl.ds(s * rw, rw)])

    return deg


_deg_n = _make_deg(ACC_N, ACC_N)
_deg_h = _make_deg(NH, ACC_H)


def _make_conv(n_out, acc_rows, k, nbuf, n_stages, table_rows=0):
    ch = EP // k
    ch2 = ch // n_stages
    lead = nbuf - 1

    @functools.partial(
        pl.kernel,
        mesh=_MESH,
        out_type=jax.ShapeDtypeStruct((2, n_out, D), jnp.float32),
        scratch_types=(
            [pltpu.VMEM((ch2, k), jnp.int32)] * 2
            + [pltpu.VMEM((k, D), jnp.float32)] * nbuf
            + [pltpu.VMEM_SHARED((acc_rows, D), jnp.float32)]
            + ([pltpu.VMEM_SHARED((table_rows, D), jnp.float32)]
               if table_rows else [])
            + [pltpu.SemaphoreType.DMA] * (2 * nbuf)
        ),
    )
    def conv(x_hbm, gidx_hbm, sidx_hbm, zeros_hbm, out_hbm, *scr):
        gidx_v, sidx_v = scr[0], scr[1]
        rows = scr[2:2 + nbuf]
        acc = scr[2 + nbuf]
        off = 3 + nbuf + (1 if table_rows else 0)
        gsem = scr[off:off + nbuf]
        ssem = scr[off + nbuf:off + 2 * nbuf]
        c = lax.axis_index("c")
        s = lax.axis_index("s")
        wid = s * 2 + c
        _zero_rows(zeros_hbm, acc, s * (acc_rows // 16), acc_rows // 16, D)
        if table_rows:
            table = scr[2 + nbuf + 1]
            tr = table_rows // 16
            pltpu.sync_copy(x_hbm.at[pl.ds(s * tr, tr)],
                            table.at[pl.ds(s * tr, tr)])
            x_src = table
        else:
            x_src = x_hbm
        plsc.subcore_barrier()

        for half in range(n_stages):
            pltpu.sync_copy(gidx_hbm.at[wid, pl.ds(half * ch2, ch2)], gidx_v)
            pltpu.sync_copy(sidx_hbm.at[wid, pl.ds(half * ch2, ch2)], sidx_v)
            for j in range(lead):
                pltpu.async_copy(x_src.at[gidx_v.at[j]], rows[j], gsem[j])

            def outer(o, carry):
                for b in range(nbuf):
                    ci = o * nbuf + b
                    nxt = ci + lead
                    bn = (b + lead) % nbuf

                    @pl.when(nxt < ch2)
                    def _():
                        @pl.when(nxt >= nbuf)
                        def _():
                            pltpu.make_async_copy(
                                rows[bn], acc.at[sidx_v.at[0]],
                                ssem[bn]).wait()

                        pltpu.async_copy(x_src.at[gidx_v.at[nxt]],
                                         rows[bn], gsem[bn])

                    pltpu.make_async_copy(x_src.at[gidx_v.at[ci]],
                                          rows[b], gsem[b]).wait()
                    pltpu.async_copy(rows[b], acc.at[sidx_v.at[ci]],
                                     ssem[b], add=True)
                return carry

            lax.fori_loop(0, ch2 // nbuf, outer, 0)
            for j in range(nbuf):
                pltpu.make_async_copy(rows[j], acc.at[sidx_v.at[0]],
                                      ssem[j]).wait()
        plsc.subcore_barrier()
        rw = n_out // 16
        pltpu.sync_copy(acc.at[pl.ds(s * rw, rw)],
                        out_hbm.at[c, pl.ds(s * rw, rw)])

    return conv


K_N = 64
_conv_to_hyper = _make_conv(NH, ACC_H, K_N, 2, 4, table_rows=ACC_N)
_conv_to_node = _make_conv(ACC_N, ACC_N, K_N, 2, 4, table_rows=NH)


def _ln(x, g, b):
    mu = jnp.mean(x, axis=-1, keepdims=True)
    var = jnp.mean((x - mu) ** 2, axis=-1, keepdims=True)
    return (x - mu) / jnp.sqrt(var + EPS) * g + b


def _k0n_body(deg_ref, h_ref, cn_ref, hpre_ref):
    d = deg_ref[0] + deg_ref[1]
    cb = jax.lax.rsqrt(jnp.maximum(d, 1.0))
    cn_ref[...] = cb
    hpre_ref[...] = h_ref[...] * cb


_BN = 400


def _k0n(deg_n_p, h):
    grid = NN // _BN
    return pl.pallas_call(
        _k0n_body,
        grid=(grid,),
        in_specs=[
            pl.BlockSpec((2, _BN, D), lambda i: (0, i, 0)),
            pl.BlockSpec((_BN, D), lambda i: (i, 0)),
        ],
        out_specs=[
            pl.BlockSpec((_BN, D), lambda i: (i, 0)),
            pl.BlockSpec((_BN, D), lambda i: (i, 0)),
        ],
        out_shape=[
            jax.ShapeDtypeStruct((NN, D), jnp.float32),
            jax.ShapeDtypeStruct((ACC_N, D), jnp.float32),
        ],
    )(deg_n_p, h)


def _k0h_body(deg_ref, ch_ref):
    d = deg_ref[0] + deg_ref[1]
    ch_ref[...] = jax.lax.rsqrt(jnp.maximum(d, 1.0))


def _k0h(deg_h_p):
    return pl.pallas_call(
        _k0h_body,
        grid=(NH // 256,),
        in_specs=[pl.BlockSpec((2, 256, D), lambda i: (0, i, 0))],
        out_specs=pl.BlockSpec((256, D), lambda i: (i, 0)),
        out_shape=jax.ShapeDtypeStruct((NH, D), jnp.float32),
    )(deg_h_p)


def _k1_body(u_ref, ch_ref, w_ref, b_ref, out_ref):
    ch = ch_ref[...]
    u = (u_ref[0] + u_ref[1]) * ch
    h1 = jnp.dot(u, w_ref[...], preferred_element_type=jnp.float32) + b_ref[...]
    out_ref[...] = h1 * ch


def _k1(u_p, ch_b, w, b):
    return pl.pallas_call(
        _k1_body,
        grid=(1,),
        in_specs=[
            pl.BlockSpec((2, NH, D), lambda i: (0, 0, 0)),
            pl.BlockSpec((NH, D), lambda i: (0, 0)),
            pl.BlockSpec((D, D), lambda i: (0, 0)),
            pl.BlockSpec((1, D), lambda i: (0, 0)),
        ],
        out_specs=pl.BlockSpec((NH, D), lambda i: (0, 0)),
        out_shape=jax.ShapeDtypeStruct((NH, D), jnp.float32),
    )(u_p, ch_b, w, b)


def _k2_body(final, v_ref, cn_ref, h_ref, w2_ref, b2_ref, g1_ref, e1_ref,
             wf1_ref, bf1_ref, wf2_ref, bf2_ref, g2_ref, e2_ref,
             gf_ref, ef_ref, wo_ref, bo_ref, out1_ref, out2_ref):
    cn = cn_ref[...]
    hn = jnp.dot((v_ref[0] + v_ref[1]) * cn, w2_ref[...],
                 preferred_element_type=jnp.float32) + b2_ref[...]
    t = _ln(h_ref[...] + hn, g1_ref[...], e1_ref[...])
    f = jnp.maximum(
        jnp.dot(t, wf1_ref[...], preferred_element_type=jnp.float32)
        + bf1_ref[...], 0.0)
    f = jnp.dot(f, wf2_ref[...], preferred_element_type=jnp.float32) + bf2_ref[...]
    h2 = _ln(t + f, g2_ref[...], e2_ref[...])
    if final:
        hf = _ln(h2, gf_ref[...], ef_ref[...])
        out1_ref[...] = jnp.dot(hf, wo_ref[...],
                                preferred_element_type=jnp.float32) + bo_ref[...]
        out2_ref[...] = h2
    else:
        out1_ref[...] = h2
        out2_ref[...] = h2 * cn


def _k2(final, v_p, cn_b, h, w2, b2, g1, e1, wf1, bf1, wf2, bf2, g2, e2,
        gf, ef, wo, bo):
    grid = NN // _BN
    full = lambda shape: pl.BlockSpec(shape, lambda i: tuple(0 for _ in shape))
    return pl.pallas_call(
        functools.partial(_k2_body, final),
        grid=(grid,),
        in_specs=[
            pl.BlockSpec((2, _BN, D), lambda i: (0, i, 0)),
            pl.BlockSpec((_BN, D), lambda i: (i, 0)),
            pl.BlockSpec((_BN, D), lambda i: (i, 0)),
            full((D, D)), full((1, D)), full((1, D)), full((1, D)),
            full((D, FF)), full((1, FF)), full((FF, D)), full((1, D)),
            full((1, D)), full((1, D)),
            full((1, D)), full((1, D)), full((D, D)), full((1, D)),
        ],
        out_specs=[
            pl.BlockSpec((_BN, D), lambda i: (i, 0)),
            pl.BlockSpec((_BN, D), lambda i: (i, 0)),
        ],
        out_shape=[
            jax.ShapeDtypeStruct((NN, D), jnp.float32),
            jax.ShapeDtypeStruct((ACC_N, D), jnp.float32),
        ],
    )(v_p, cn_b, h, w2, b2, g1, e1, wf1, bf1, wf2, bf2, g2, e2, gf, ef, wo, bo)


def kernel(node_features, src_idx, dst_idx, params):
    src = src_idx.astype(jnp.int32)
    dst = dst_idx.astype(jnp.int32)
    padlen = NE_PAD - NE

    def _prep(a, fill, k):
        return jnp.concatenate(
            [a, jnp.full((padlen,), fill, jnp.int32)]).reshape(NW, EP // k, k)

    src_s128 = _prep(src, NN, K)
    dst_s128 = _prep(dst, NH, K)
    src_g = _prep(src, 0, K_N)
    dst_s = _prep(dst, NH, K_N)
    dst_g = _prep(dst, 0, K_N)
    src_s = _prep(src, NN, K_N)

    zeros_rows = jnp.zeros((K, D), jnp.float32)
    ones_rows = jnp.ones((K, D), jnp.float32)

    deg_n_p = _deg_n(src_s128, zeros_rows, ones_rows)
    deg_h_p = _deg_h(dst_s128, zeros_rows, ones_rows)
    cn_b, h_pre = _k0n(deg_n_p, node_features)
    ch_b = _k0h(deg_h_p)

    h = node_features
    out = None
    rb = lambda x: x.reshape(1, -1)
    for l, p in enumerate(params['layers']):
        final = l == len(params['layers']) - 1
        u_p = _conv_to_hyper(h_pre, src_g, dst_s, zeros_rows)
        u2 = _k1(u_p, ch_b, p['W_n2h'], rb(p['b_n2h']))
        v_p = _conv_to_node(u2, dst_g, src_s, zeros_rows)
        out1, out2 = _k2(final, v_p, cn_b, h,
                         p['W_h2n'], rb(p['b_h2n']),
                         rb(p['ln1_g']), rb(p['ln1_b']),
                         p['W_ff1'], rb(p['b_ff1']),
                         p['W_ff2'], rb(p['b_ff2']),
                         rb(p['ln2_g']), rb(p['ln2_b']),
                         rb(params['ln_f_g']), rb(params['ln_f_b']),
                         params['W_out'], rb(params['b_out']))
        if final:
            out = out1
        else:
            h, h_pre = out1, out2
    return out


# speedup vs baseline: 7.2703x; 1.0047x over previous
import functools

import jax
import jax.numpy as jnp
from jax import lax
from jax.experimental import pallas as pl
from jax.experimental.pallas import tpu as pltpu
from jax.experimental.pallas import tpu_sc as plsc

NN = 10000
NH = 2048
NE = 320000
D = 128
FF = 4 * D
EPS = 1e-5

NW = 32
K = 128
EP = 10240
CH = EP // K
NE_PAD = NW * EP
ACC_N = 10240
ACC_H = 2560

_MESH = plsc.VectorSubcoreMesh(core_axis_name="c", subcore_axis_name="s",
                               num_cores=2, num_subcores=16)


def _zero_rows(zeros_hbm, acc, base, nrows, width):
    off = 0
    while off < nrows:
        step = min(K, nrows - off)
        pltpu.sync_copy(zeros_hbm.at[pl.ds(0, step)],
                        acc.at[pl.ds(base + off, step)])
        off += step


@functools.partial(
    pl.kernel,
    mesh=_MESH,
    out_type=[
        jax.ShapeDtypeStruct((ACC_N, D), jnp.float32),
        jax.ShapeDtypeStruct((NH, D), jnp.float32),
    ],
    scratch_types=[
        pltpu.VMEM((CH // 2, K), jnp.int32),
        pltpu.VMEM((K, D), jnp.float32),
        pltpu.VMEM_SHARED((ACC_N, D), jnp.float32),
        pltpu.SemaphoreType.DMA,
    ],
)
def _deg_both(src_s_hbm, dst_s_hbm, zeros_hbm, ones_hbm,
              out_n, out_h, sidx_v, ones_v, acc, sem):
    c = lax.axis_index("c")
    s = lax.axis_index("s")
    ch2 = CH // 2

    @pl.when(c == 0)
    def _():
        _zero_rows(zeros_hbm, acc, s * (ACC_H // 16), ACC_H // 16, D)

    @pl.when(c == 1)
    def _():
        _zero_rows(zeros_hbm, acc, s * (ACC_N // 16), ACC_N // 16, D)

    pltpu.sync_copy(ones_hbm, ones_v)
    plsc.subcore_barrier()

    def scatter_all(idx_hbm):
        for part in (0, 1):
            for half in (0, 1):
                pltpu.sync_copy(
                    idx_hbm.at[2 * s + part, pl.ds(half * ch2, ch2)], sidx_v)

                def body(ci, carry):
                    pltpu.async_copy(ones_v, acc.at[sidx_v.at[ci]],
                                     sem, add=True)
                    return carry

                lax.fori_loop(0, ch2, body, 0)

                def drain(ci, carry):
                    pltpu.make_async_copy(ones_v, acc.at[sidx_v.at[ci]],
                                          sem).wait()
                    return carry

                lax.fori_loop(0, ch2, drain, 0)

    @pl.when(c == 0)
    def _():
        scatter_all(dst_s_hbm)

    @pl.when(c == 1)
    def _():
        scatter_all(src_s_hbm)

    plsc.subcore_barrier()

    @pl.when(c == 0)
    def _():
        rw = NH // 16
        pltpu.sync_copy(acc.at[pl.ds(s * rw, rw)], out_h.at[pl.ds(s * rw, rw)])

    @pl.when(c == 1)
    def _():
        rw = ACC_N // 16
        pltpu.sync_copy(acc.at[pl.ds(s * rw, rw)], out_n.at[pl.ds(s * rw, rw)])


def _make_conv(n_out, acc_rows, k, nbuf, n_stages, table_rows=0):
    ch = EP // k
    ch2 = ch // n_stages
    lead = nbuf - 1

    @functools.partial(
        pl.kernel,
        mesh=_MESH,
        out_type=jax.ShapeDtypeStruct((2, n_out, D), jnp.float32),
        scratch_types=(
            [pltpu.VMEM((ch2, k), jnp.int32)] * 2
            + [pltpu.VMEM((k, D), jnp.float32)] * nbuf
            + [pltpu.VMEM_SHARED((acc_rows, D), jnp.float32)]
            + ([pltpu.VMEM_SHARED((table_rows, D), jnp.float32)]
               if table_rows else [])
            + [pltpu.SemaphoreType.DMA] * (2 * nbuf)
        ),
    )
    def conv(x_hbm, gidx_hbm, sidx_hbm, zeros_hbm, out_hbm, *scr):
        gidx_v, sidx_v = scr[0], scr[1]
        rows = scr[2:2 + nbuf]
        acc = scr[2 + nbuf]
        off = 3 + nbuf + (1 if table_rows else 0)
        gsem = scr[off:off + nbuf]
        ssem = scr[off + nbuf:off + 2 * nbuf]
        c = lax.axis_index("c")
        s = lax.axis_index("s")
        wid = s * 2 + c
        _zero_rows(zeros_hbm, acc, s * (acc_rows // 16), acc_rows // 16, D)
        if table_rows:
            table = scr[2 + nbuf + 1]
            tr = table_rows // 16
            pltpu.sync_copy(x_hbm.at[pl.ds(s * tr, tr)],
                            table.at[pl.ds(s * tr, tr)])
            x_src = table
        else:
            x_src = x_hbm
        plsc.subcore_barrier()

        for half in range(n_stages):
            pltpu.sync_copy(gidx_hbm.at[wid, pl.ds(half * ch2, ch2)], gidx_v)
            pltpu.sync_copy(sidx_hbm.at[wid, pl.ds(half * ch2, ch2)], sidx_v)
            for j in range(lead):
                pltpu.async_copy(x_src.at[gidx_v.at[j]], rows[j], gsem[j])

            def outer(o, carry):
                for b in range(nbuf):
                    ci = o * nbuf + b
                    nxt = ci + lead
                    bn = (b + lead) % nbuf

                    @pl.when(nxt < ch2)
                    def _():
                        @pl.when(nxt >= nbuf)
                        def _():
                            pltpu.make_async_copy(
                                rows[bn], acc.at[sidx_v.at[0]],
                                ssem[bn]).wait()

                        pltpu.async_copy(x_src.at[gidx_v.at[nxt]],
                                         rows[bn], gsem[bn])

                    pltpu.make_async_copy(x_src.at[gidx_v.at[ci]],
                                          rows[b], gsem[b]).wait()
                    pltpu.async_copy(rows[b], acc.at[sidx_v.at[ci]],
                                     ssem[b], add=True)
                return carry

            lax.fori_loop(0, ch2 // nbuf, outer, 0)
            for j in range(nbuf):
                pltpu.make_async_copy(rows[j], acc.at[sidx_v.at[0]],
                                      ssem[j]).wait()
        plsc.subcore_barrier()
        rw = n_out // 16
        pltpu.sync_copy(acc.at[pl.ds(s * rw, rw)],
                        out_hbm.at[c, pl.ds(s * rw, rw)])

    return conv


K_N = 64
_conv_to_hyper = _make_conv(NH, ACC_H, K_N, 2, 4, table_rows=ACC_N)
_conv_to_node = _make_conv(ACC_N, ACC_N, K_N, 2, 4, table_rows=NH)


def _ln(x, g, b):
    mu = jnp.mean(x, axis=-1, keepdims=True)
    var = jnp.mean((x - mu) ** 2, axis=-1, keepdims=True)
    return (x - mu) / jnp.sqrt(var + EPS) * g + b


def _k0n_body(deg_ref, h_ref, cn_ref, hpre_ref):
    cb = jax.lax.rsqrt(jnp.maximum(deg_ref[...], 1.0))
    cn_ref[...] = cb
    hpre_ref[...] = h_ref[...] * cb


_BN = 400


def _k0n(deg_n_p, h):
    grid = NN // _BN
    return pl.pallas_call(
        _k0n_body,
        grid=(grid,),
        in_specs=[
            pl.BlockSpec((_BN, D), lambda i: (i, 0)),
            pl.BlockSpec((_BN, D), lambda i: (i, 0)),
        ],
        out_specs=[
            pl.BlockSpec((_BN, D), lambda i: (i, 0)),
            pl.BlockSpec((_BN, D), lambda i: (i, 0)),
        ],
        out_shape=[
            jax.ShapeDtypeStruct((NN, D), jnp.float32),
            jax.ShapeDtypeStruct((ACC_N, D), jnp.float32),
        ],
    )(deg_n_p, h)


def _k0h_body(deg_ref, ch_ref):
    ch_ref[...] = jax.lax.rsqrt(jnp.maximum(deg_ref[...], 1.0))


def _k0h(deg_h_p):
    return pl.pallas_call(
        _k0h_body,
        grid=(NH // 256,),
        in_specs=[pl.BlockSpec((256, D), lambda i: (i, 0))],
        out_specs=pl.BlockSpec((256, D), lambda i: (i, 0)),
        out_shape=jax.ShapeDtypeStruct((NH, D), jnp.float32),
    )(deg_h_p)


def _k1_body(u_ref, ch_ref, w_ref, b_ref, out_ref):
    ch = ch_ref[...]
    u = (u_ref[0] + u_ref[1]) * ch
    h1 = jnp.dot(u, w_ref[...], preferred_element_type=jnp.float32) + b_ref[...]
    out_ref[...] = h1 * ch


def _k1(u_p, ch_b, w, b):
    return pl.pallas_call(
        _k1_body,
        grid=(1,),
        in_specs=[
            pl.BlockSpec((2, NH, D), lambda i: (0, 0, 0)),
            pl.BlockSpec((NH, D), lambda i: (0, 0)),
            pl.BlockSpec((D, D), lambda i: (0, 0)),
            pl.BlockSpec((1, D), lambda i: (0, 0)),
        ],
        out_specs=pl.BlockSpec((NH, D), lambda i: (0, 0)),
        out_shape=jax.ShapeDtypeStruct((NH, D), jnp.float32),
    )(u_p, ch_b, w, b)


def _k2_body(final, v_ref, cn_ref, h_ref, w2_ref, b2_ref, g1_ref, e1_ref,
             wf1_ref, bf1_ref, wf2_ref, bf2_ref, g2_ref, e2_ref,
             gf_ref, ef_ref, wo_ref, bo_ref, out1_ref, out2_ref):
    cn = cn_ref[...]
    hn = jnp.dot((v_ref[0] + v_ref[1]) * cn, w2_ref[...],
                 preferred_element_type=jnp.float32) + b2_ref[...]
    t = _ln(h_ref[...] + hn, g1_ref[...], e1_ref[...])
    f = jnp.maximum(
        jnp.dot(t, wf1_ref[...], preferred_element_type=jnp.float32)
        + bf1_ref[...], 0.0)
    f = jnp.dot(f, wf2_ref[...], preferred_element_type=jnp.float32) + bf2_ref[...]
    h2 = _ln(t + f, g2_ref[...], e2_ref[...])
    if final:
        hf = _ln(h2, gf_ref[...], ef_ref[...])
        out1_ref[...] = jnp.dot(hf, wo_ref[...],
                                preferred_element_type=jnp.float32) + bo_ref[...]
        out2_ref[...] = h2
    else:
        out1_ref[...] = h2
        out2_ref[...] = h2 * cn


def _k2(final, v_p, cn_b, h, w2, b2, g1, e1, wf1, bf1, wf2, bf2, g2, e2,
        gf, ef, wo, bo):
    grid = NN // _BN
    full = lambda shape: pl.BlockSpec(shape, lambda i: tuple(0 for _ in shape))
    return pl.pallas_call(
        functools.partial(_k2_body, final),
        grid=(grid,),
        in_specs=[
            pl.BlockSpec((2, _BN, D), lambda i: (0, i, 0)),
            pl.BlockSpec((_BN, D), lambda i: (i, 0)),
            pl.BlockSpec((_BN, D), lambda i: (i, 0)),
            full((D, D)), full((1, D)), full((1, D)), full((1, D)),
            full((D, FF)), full((1, FF)), full((FF, D)), full((1, D)),
            full((1, D)), full((1, D)),
            full((1, D)), full((1, D)), full((D, D)), full((1, D)),
        ],
        out_specs=[
            pl.BlockSpec((_BN, D), lambda i: (i, 0)),
            pl.BlockSpec((_BN, D), lambda i: (i, 0)),
        ],
        out_shape=[
            jax.ShapeDtypeStruct((NN, D), jnp.float32),
            jax.ShapeDtypeStruct((ACC_N, D), jnp.float32),
        ],
    )(v_p, cn_b, h, w2, b2, g1, e1, wf1, bf1, wf2, bf2, g2, e2, gf, ef, wo, bo)


def kernel(node_features, src_idx, dst_idx, params):
    src = src_idx.astype(jnp.int32)
    dst = dst_idx.astype(jnp.int32)
    padlen = NE_PAD - NE

    def _prep(a, fill, k):
        return jnp.concatenate(
            [a, jnp.full((padlen,), fill, jnp.int32)]).reshape(NW, EP // k, k)

    src_s128 = _prep(src, NN, K)
    dst_s128 = _prep(dst, NH, K)
    src_g = _prep(src, 0, K_N)
    dst_s = _prep(dst, NH, K_N)
    dst_g = _prep(dst, 0, K_N)
    src_s = _prep(src, NN, K_N)

    zeros_rows = jnp.zeros((K, D), jnp.float32)
    ones_rows = jnp.ones((K, D), jnp.float32)

    deg_n_p, deg_h_p = _deg_both(src_s128, dst_s128, zeros_rows, ones_rows)
    cn_b, h_pre = _k0n(deg_n_p, node_features)
    ch_b = _k0h(deg_h_p)

    h = node_features
    out = None
    rb = lambda x: x.reshape(1, -1)
    for l, p in enumerate(params['layers']):
        final = l == len(params['layers']) - 1
        u_p = _conv_to_hyper(h_pre, src_g, dst_s, zeros_rows)
        u2 = _k1(u_p, ch_b, p['W_n2h'], rb(p['b_n2h']))
        v_p = _conv_to_node(u2, dst_g, src_s, zeros_rows)
        out1, out2 = _k2(final, v_p, cn_b, h,
                         p['W_h2n'], rb(p['b_h2n']),
                         rb(p['ln1_g']), rb(p['ln1_b']),
                         p['W_ff1'], rb(p['b_ff1']),
                         p['W_ff2'], rb(p['b_ff2']),
                         rb(p['ln2_g']), rb(p['ln2_b']),
                         rb(params['ln_f_g']), rb(params['ln_f_b']),
                         params['W_out'], rb(params['b_out']))
        if final:
            out = out1
        else:
            h, h_pre = out1, out2
    return out
